# baseline (speedup 1.0000x reference)
import functools
import math

import jax
import jax.numpy as jnp
from jax import lax
from jax.experimental import pallas as pl
from jax.experimental.pallas import tpu as pltpu
from jax.experimental.pallas import tpu_sc as plsc

B = 4
S = 2048
H = 2048
D = 2048
E = 8

_TS = 256
_TH = 128
_TM = 512

_INV_SQRT2 = 1.0 / math.sqrt(2.0)

_NC = 2
_NS = 16
_NW = _NC * _NS
_LANES = 16
_ROWS_PER_W = D // _NW
_CH = 2
_NCH = _ROWS_PER_W // _CH
_NBUF = 2


def _router_body(x_ref, w_ref, b_ref, rw_ref, crep_ref, acc_ref):
    i = pl.program_id(0)

    @pl.when(i == 0)
    def _init():
        acc_ref[...] = jnp.zeros_like(acc_ref)

    acc_ref[...] += jnp.sum(x_ref[...], axis=1)

    @pl.when(i == pl.num_programs(0) - 1)
    def _finish():
        pooled = acc_ref[...] * (1.0 / S)
        logits = jnp.dot(pooled, w_ref[...], preferred_element_type=jnp.float32)
        logits = logits + b_ref[...]
        m = jnp.max(logits, axis=-1, keepdims=True)
        p = jnp.exp(logits - m)
        rw = p / jnp.sum(p, axis=-1, keepdims=True)
        rw_ref[...] = rw
        coef = jnp.sum(rw, axis=0)
        crep_ref[...] = lax.broadcast_in_dim(coef, (E, 128), (0,))


def _router(x, router_W, router_b):
    return pl.pallas_call(
        _router_body,
        grid=(S // _TS,),
        in_specs=[
            pl.BlockSpec((B, _TS, H), lambda i: (0, i, 0)),
            pl.BlockSpec((H, E), lambda i: (0, 0)),
            pl.BlockSpec((1, E), lambda i: (0, 0)),
        ],
        out_specs=[
            pl.BlockSpec((B, E), lambda i: (0, 0)),
            pl.BlockSpec((E, 128), lambda i: (0, 0)),
        ],
        out_shape=[
            jax.ShapeDtypeStruct((B, E), jnp.float32),
            jax.ShapeDtypeStruct((E, 128), jnp.float32),
        ],
        scratch_shapes=[pltpu.VMEM((B, H), jnp.float32)],
        compiler_params=pltpu.CompilerParams(
            dimension_semantics=("arbitrary",)),
    )(x, router_W, router_b.reshape(1, E))


def _merge_down_body(rw_ref, dw_ref, db_ref, wd_ref, bd_ref):
    rw = rw_ref[...]
    acc = None
    for e in range(E):
        c = jnp.sum(rw[:, e])
        t = c * dw_ref[e]
        acc = t if acc is None else acc + t
    wd_ref[...] = acc.astype(jnp.bfloat16)

    @pl.when(pl.program_id(0) == 0)
    def _bias():
        bacc = None
        for e in range(E):
            c = jnp.sum(rw[:, e])
            t = c * db_ref[e:e + 1, :]
            bacc = t if bacc is None else bacc + t
        bd_ref[...] = bacc


def _merge_down(rw, down_W, down_b):
    return pl.pallas_call(
        _merge_down_body,
        grid=(H // _TH,),
        in_specs=[
            pl.BlockSpec((B, E), lambda i: (0, 0)),
            pl.BlockSpec((E, _TH, D), lambda i: (0, i, 0)),
            pl.BlockSpec((E, D), lambda i: (0, 0)),
        ],
        out_specs=[
            pl.BlockSpec((_TH, D), lambda i: (i, 0)),
            pl.BlockSpec((1, D), lambda i: (0, 0)),
        ],
        out_shape=[
            jax.ShapeDtypeStruct((H, D), jnp.bfloat16),
            jax.ShapeDtypeStruct((1, D), jnp.float32),
        ],
        compiler_params=pltpu.CompilerParams(
            dimension_semantics=("arbitrary",)),
    )(rw, down_W, down_b)


def _merge_up_sc_body(up_hbm, crep_hbm, out_hbm, crep_v, bufs, acc, sem0, sem1):
    core = lax.axis_index("c")
    sub = lax.axis_index("s")
    wid = sub * _NC + core
    row0 = wid * _ROWS_PER_W

    pltpu.sync_copy(crep_hbm, crep_v)
    ce = [crep_v[e, pl.ds(0, _LANES)] for e in range(E)]

    def fire(ch_idx, b, sem):
        r = row0 + ch_idx * _CH
        for e in range(E):
            pltpu.async_copy(up_hbm.at[e, pl.ds(r, _CH), :], bufs.at[b, e], sem)

    def drain(b, sem):
        pltpu.make_async_copy(
            up_hbm.at[:, pl.ds(0, _CH), :], bufs.at[b], sem).wait()

    def compute_store(ch_idx, b):
        r = row0 + ch_idx * _CH
        for rr in range(_CH):
            def lane_body(j, _):
                sl = pl.ds(j * _LANES, _LANES)
                v = ce[0] * bufs[b, 0, rr, sl]
                for e in range(1, E):
                    v = v + ce[e] * bufs[b, e, rr, sl]
                acc[rr, sl] = v
                return _
            lax.fori_loop(0, H // _LANES, lane_body, 0)
        pltpu.sync_copy(acc, out_hbm.at[pl.ds(r, _CH), :])

    sems = (sem0, sem1)
    fire(0, 0, sems[0])

    def outer(c, _):
        for b in range(_NBUF):
            ch = c + b
            nxt = ch + 1

            @pl.when(nxt < _NCH)
            def _prefetch():
                fire(nxt, 1 - b, sems[1 - b])

            drain(b, sems[b])
            compute_store(ch, b)
        return _

    lax.fori_loop(0, _NCH // _NBUF, lambda i, _: outer(i * _NBUF, _), 0)


def _merge_up_sc(up_W, crep):
    mesh = plsc.VectorSubcoreMesh(core_axis_name="c", subcore_axis_name="s")
    run = functools.partial(
        pl.kernel,
        mesh=mesh,
        out_type=jax.ShapeDtypeStruct((D, H), jnp.float32),
        scratch_types=[
            pltpu.VMEM((E, 128), jnp.float32),
            pltpu.VMEM((_NBUF, E, _CH, H), jnp.float32),
            pltpu.VMEM((_CH, H), jnp.float32),
            pltpu.SemaphoreType.DMA,
            pltpu.SemaphoreType.DMA,
        ],
    )(_merge_up_sc_body)
    return run(up_W, crep)


def _mm1_body(x_ref, wd_ref, bd_ref, z_ref):
    xb = x_ref[...].astype(jnp.bfloat16)
    z = jnp.dot(xb, wd_ref[...], preferred_element_type=jnp.float32)
    z = z + bd_ref[...]
    z = 0.5 * z * (1.0 + jax.lax.erf(z * _INV_SQRT2))
    z_ref[...] = z.astype(jnp.bfloat16)


def _mm1(x2d, wd, bd):
    M = x2d.shape[0]
    return pl.pallas_call(
        _mm1_body,
        grid=(M // _TM,),
        in_specs=[
            pl.BlockSpec((_TM, H), lambda i: (i, 0)),
            pl.BlockSpec((H, D), lambda i: (0, 0)),
            pl.BlockSpec((1, D), lambda i: (0, 0)),
        ],
        out_specs=pl.BlockSpec((_TM, D), lambda i: (i, 0)),
        out_shape=jax.ShapeDtypeStruct((M, D), jnp.bfloat16),
        compiler_params=pltpu.CompilerParams(
            dimension_semantics=("arbitrary",)),
    )(x2d, wd, bd)


def _mm2_body(z_ref, wu_ref, out_ref):
    wub = wu_ref[...].astype(jnp.bfloat16)
    out_ref[...] = jnp.dot(z_ref[...], wub, preferred_element_type=jnp.float32)


def _mm2(z, wu):
    M = z.shape[0]
    return pl.pallas_call(
        _mm2_body,
        grid=(M // _TM,),
        in_specs=[
            pl.BlockSpec((_TM, D), lambda i: (i, 0)),
            pl.BlockSpec((D, H), lambda i: (0, 0)),
        ],
        out_specs=pl.BlockSpec((_TM, H), lambda i: (i, 0)),
        out_shape=jax.ShapeDtypeStruct((M, H), jnp.float32),
        compiler_params=pltpu.CompilerParams(
            dimension_semantics=("arbitrary",)),
    )(z, wu)


def kernel(x, router_W, router_b, down_W, down_b, up_W):
    rw, crep = _router(x, router_W, router_b)
    wu = _merge_up_sc(up_W, crep)
    wd, bd = _merge_down(rw, down_W, down_b)
    x2d = x.reshape(B * S, H)
    z = _mm1(x2d, wd, bd)
    out = _mm2(z, wu)
    return out.reshape(B, S, H), rw


# speedup vs baseline: 1.1643x; 1.1643x over previous
import math

import jax
import jax.numpy as jnp
from jax import lax
from jax.experimental import pallas as pl
from jax.experimental.pallas import tpu as pltpu

B = 4
S = 2048
H = 2048
D = 2048
E = 8

_TS = 256
_NR = S // _TS
_TH = 128
_NMD = H // _TH
_TM = 512
_NT = B * S // _TM
_UPCH = H // _NT

_INV_SQRT2 = 1.0 / math.sqrt(2.0)


def _head_body(x_ref, w_ref, b_ref, dw_ref, db_ref,
               rw_ref, crep_ref, wd_ref, bd_ref, acc_ref, coef_ref):
    i = pl.program_id(0)

    @pl.when(i < _NR)
    def _router_phase():
        @pl.when(i == 0)
        def _init():
            acc_ref[...] = jnp.zeros_like(acc_ref)

        acc_ref[...] += jnp.sum(x_ref[...], axis=1)

        @pl.when(i == _NR - 1)
        def _finish():
            pooled = acc_ref[...] * (1.0 / S)
            logits = jnp.dot(pooled, w_ref[...],
                             preferred_element_type=jnp.float32)
            logits = logits + b_ref[...]
            m = jnp.max(logits, axis=-1, keepdims=True)
            p = jnp.exp(logits - m)
            rw = p / jnp.sum(p, axis=-1, keepdims=True)
            rw_ref[...] = rw
            coef = jnp.sum(rw, axis=0)
            crep_ref[...] = lax.broadcast_in_dim(coef, (E, 128), (0,))
            coef_ref[...] = coef.reshape(1, E)
            bacc = None
            for e in range(E):
                t = coef[e] * db_ref[e:e + 1, :]
                bacc = t if bacc is None else bacc + t
            bd_ref[...] = bacc

    @pl.when(i >= _NR)
    def _merge_down_phase():
        acc = None
        for e in range(E):
            c = coef_ref[0, e]
            t = c * dw_ref[e]
            acc = t if acc is None else acc + t
        wd_ref[...] = acc.astype(jnp.bfloat16)


def _head(x, router_W, router_b, down_W, down_b):
    return pl.pallas_call(
        _head_body,
        grid=(_NR + _NMD,),
        in_specs=[
            pl.BlockSpec((B, _TS, H),
                         lambda i: (0, jnp.minimum(i, _NR - 1), 0)),
            pl.BlockSpec((H, E), lambda i: (0, 0)),
            pl.BlockSpec((1, E), lambda i: (0, 0)),
            pl.BlockSpec((E, _TH, D),
                         lambda i: (0, jnp.clip(i - _NR, 0, _NMD - 1), 0)),
            pl.BlockSpec((E, D), lambda i: (0, 0)),
        ],
        out_specs=[
            pl.BlockSpec((B, E), lambda i: (0, 0)),
            pl.BlockSpec((E, 128), lambda i: (0, 0)),
            pl.BlockSpec((_TH, D),
                         lambda i: (jnp.clip(i - _NR, 0, _NMD - 1), 0)),
            pl.BlockSpec((1, D), lambda i: (0, 0)),
        ],
        out_shape=[
            jax.ShapeDtypeStruct((B, E), jnp.float32),
            jax.ShapeDtypeStruct((E, 128), jnp.float32),
            jax.ShapeDtypeStruct((H, D), jnp.bfloat16),
            jax.ShapeDtypeStruct((1, D), jnp.float32),
        ],
        scratch_shapes=[
            pltpu.VMEM((B, H), jnp.float32),
            pltpu.VMEM((1, E), jnp.float32),
        ],
        compiler_params=pltpu.CompilerParams(
            dimension_semantics=("arbitrary",)),
    )(x, router_W, router_b.reshape(1, E), down_W, down_b)


def _mm1_merge_body(x_ref, wd_ref, bd_ref, crep_ref, up_hbm,
                    z_ref, wu_ref, stg, sem):
    k = pl.program_id(0)
    nsteps = pl.num_programs(0)

    def chunk_copy(c, slot):
        return pltpu.make_async_copy(
            up_hbm.at[:, pl.ds(c * _UPCH, _UPCH), :], stg.at[slot],
            sem.at[slot])

    @pl.when(k == 0)
    def _prime():
        chunk_copy(0, 0).start()

    @pl.when(k + 1 < nsteps)
    def _prefetch():
        chunk_copy(k + 1, (k + 1) % 2).start()

    xb = x_ref[...].astype(jnp.bfloat16)
    z = jnp.dot(xb, wd_ref[...], preferred_element_type=jnp.float32)
    z = z + bd_ref[...]
    z = 0.5 * z * (1.0 + jax.lax.erf(z * _INV_SQRT2))
    z_ref[...] = z.astype(jnp.bfloat16)

    chunk_copy(k, k % 2).wait()
    slot = k % 2
    acc = None
    for e in range(E):
        c = crep_ref[e, 0]
        t = c * stg[slot, e]
        acc = t if acc is None else acc + t
    wu_ref[pl.ds(k * _UPCH, _UPCH), :] = acc.astype(jnp.bfloat16)


def _mm1_merge(x2d, wd, bd, crep, up_W):
    M = x2d.shape[0]
    return pl.pallas_call(
        _mm1_merge_body,
        grid=(M // _TM,),
        in_specs=[
            pl.BlockSpec((_TM, H), lambda i: (i, 0)),
            pl.BlockSpec((H, D), lambda i: (0, 0)),
            pl.BlockSpec((1, D), lambda i: (0, 0)),
            pl.BlockSpec((E, 128), lambda i: (0, 0)),
            pl.BlockSpec(memory_space=pl.ANY),
        ],
        out_specs=[
            pl.BlockSpec((_TM, D), lambda i: (i, 0)),
            pl.BlockSpec((D, H), lambda i: (0, 0)),
        ],
        out_shape=[
            jax.ShapeDtypeStruct((M, D), jnp.bfloat16),
            jax.ShapeDtypeStruct((D, H), jnp.bfloat16),
        ],
        scratch_shapes=[
            pltpu.VMEM((2, E, _UPCH, H), jnp.float32),
            pltpu.SemaphoreType.DMA((2,)),
        ],
        compiler_params=pltpu.CompilerParams(
            dimension_semantics=("arbitrary",)),
    )(x2d, wd, bd, crep, up_W)


def _mm2_body(z_ref, wu_ref, out_ref):
    out_ref[...] = jnp.dot(z_ref[...], wu_ref[...],
                           preferred_element_type=jnp.float32)


def _mm2(z, wu):
    M = z.shape[0]
    return pl.pallas_call(
        _mm2_body,
        grid=(M // _TM,),
        in_specs=[
            pl.BlockSpec((_TM, D), lambda i: (i, 0)),
            pl.BlockSpec((D, H), lambda i: (0, 0)),
        ],
        out_specs=pl.BlockSpec((_TM, H), lambda i: (i, 0)),
        out_shape=jax.ShapeDtypeStruct((M, H), jnp.float32),
        compiler_params=pltpu.CompilerParams(
            dimension_semantics=("arbitrary",)),
    )(z, wu)


def kernel(x, router_W, router_b, down_W, down_b, up_W):
    rw, crep, wd, bd = _head(x, router_W, router_b, down_W, down_b)
    x2d = x.reshape(B * S, H)
    z, wu = _mm1_merge(x2d, wd, bd, crep, up_W)
    out = _mm2(z, wu)
    return out.reshape(B, S, H), rw
